# baseline (speedup 1.0000x reference)
import functools

import jax
import jax.numpy as jnp
from jax import lax
from jax.experimental import pallas as pl
from jax.experimental.pallas import tpu as pltpu
from jax.experimental.pallas import tpu_sc as plsc

_NUM_CORES = 2
_NUM_SUBCORES = 16
_NUM_WORKERS = _NUM_CORES * _NUM_SUBCORES
_LANES = 16

_IDS_GROUP = 32
_DEPTH = 6
_PASS = 40


@functools.partial(jax.jit, static_argnames=("batch", "seq", "dim"))
def _emb_mean(ids_flat, table, *, batch, seq, dim):
    per_worker = batch // _NUM_WORKERS
    groups = per_worker // _IDS_GROUP
    vregs = dim // _LANES
    npass = seq // _PASS
    chunks = [(p * _PASS, _PASS) for p in range(npass)]

    mesh = plsc.VectorSubcoreMesh(core_axis_name="c", subcore_axis_name="s")

    @functools.partial(
        pl.kernel,
        out_type=jax.ShapeDtypeStruct((batch, dim), jnp.float32),
        mesh=mesh,
        scratch_types=[
            pltpu.VMEM((2, _IDS_GROUP * seq), jnp.int32),
            pltpu.VMEM((_DEPTH, _PASS, dim), jnp.float32),
            pltpu.VMEM((per_worker, dim), jnp.float32),
            pltpu.SemaphoreType.DMA,
            pltpu.SemaphoreType.DMA,
        ],
        compiler_params=pltpu.CompilerParams(use_tc_tiling_on_sc=False),
    )
    def body(ids_hbm, table_hbm, out_hbm, ids_v, rows_v, out_v, sem_g, sem_i):
        wid = lax.axis_index("s") * _NUM_CORES + lax.axis_index("c")
        base = wid * per_worker

        def ids_copy(g, par):
            grp = base + g * _IDS_GROUP
            return pltpu.make_async_copy(
                ids_hbm.at[pl.ds(grp * seq, _IDS_GROUP * seq)],
                ids_v.at[par],
                sem_i,
            )

        def issue_gathers(s, ipar, rpar):
            return [
                pltpu.async_copy(
                    table_hbm.at[ids_v.at[ipar, pl.ds(s * seq + off, n)]],
                    rows_v.at[rpar],
                    sem_g,
                    add=True,
                )
                for off, n in chunks
            ]

        def reduce_store(s, rpar, row):
            zero = jnp.zeros((_LANES,), jnp.float32)

            def red_body(t, a):
                t8 = t * 8
                for dt in range(8):
                    vs = []
                    for k in range(vregs):
                        sl = pl.ds(k * _LANES, _LANES)
                        vs.append(rows_v[rpar, t8 + dt, sl])
                        rows_v[rpar, t8 + dt, sl] = zero
                    a = tuple(a[k] + vs[k] for k in range(vregs))
                return a

            acc = lax.fori_loop(0, _PASS // 8, red_body, (zero,) * vregs)

            scale = jnp.float32(1.0 / seq)
            for k in range(vregs):
                out_v[row, pl.ds(k * _LANES, _LANES)] = acc[k] * scale

        zero16 = jnp.zeros((_LANES,), jnp.float32)

        def zero_body(t, _):
            for d in range(_DEPTH):
                for k in range(vregs):
                    rows_v[d, t, pl.ds(k * _LANES, _LANES)] = zero16
            return 0

        lax.fori_loop(0, _PASS, zero_body, 0)
        ids_copy(0, 0).start()

        def two_groups(h, _):
            for par in (0, 1):
                g = 2 * h + par
                ids_copy(g, par).wait()
                @pl.when(g + 1 < groups)
                def _():
                    ids_copy(g + 1, 1 - par).start()

                ahead = _DEPTH - 1
                cps = {
                    s: issue_gathers(s, par, s % _DEPTH)
                    for s in range(min(ahead, _IDS_GROUP))
                }
                for s in range(_IDS_GROUP):
                    if s + ahead < _IDS_GROUP:
                        cps[s + ahead] = issue_gathers(
                            s + ahead, par, (s + ahead) % _DEPTH
                        )
                    for cp in cps.pop(s):
                        cp.wait()
                    reduce_store(s, s % _DEPTH, g * _IDS_GROUP + s)
            return 0

        lax.fori_loop(0, groups // 2, two_groups, 0)
        pltpu.sync_copy(out_v, out_hbm.at[pl.ds(base, per_worker)])

    return body(ids_flat, table)


def _classifier_body(x_ref, w_ref, b_ref, o_ref):
    o_ref[...] = (
        lax.dot_general(
            x_ref[...],
            w_ref[...],
            (((1,), (1,)), ((), ())),
            preferred_element_type=jnp.float32,
        )
        + b_ref[...]
    )


def kernel(input_ids, emb_table, W, b):
    batch, seq = input_ids.shape
    _, dim = emb_table.shape
    ncls = W.shape[0]

    ids_flat = input_ids.reshape(-1).astype(jnp.int32)
    x = _emb_mean(ids_flat, emb_table, batch=batch, seq=seq, dim=dim)
    logits = pl.pallas_call(
        _classifier_body,
        out_shape=jax.ShapeDtypeStruct((batch, ncls), jnp.float32),
    )(x, W, b.reshape(1, ncls))
    return (logits, x)


# speedup vs baseline: 1.0168x; 1.0168x over previous
import functools

import jax
import jax.numpy as jnp
from jax import lax
from jax.experimental import pallas as pl
from jax.experimental.pallas import tpu as pltpu
from jax.experimental.pallas import tpu_sc as plsc

_NUM_CORES = 2
_NUM_SUBCORES = 16
_NUM_WORKERS = _NUM_CORES * _NUM_SUBCORES
_LANES = 16

_IDS_GROUP = 32
_DEPTH = 4
_PASS = 40


@functools.partial(jax.jit, static_argnames=("batch", "seq", "dim"))
def _emb_mean(ids_flat, table, *, batch, seq, dim):
    per_worker = batch // _NUM_WORKERS
    groups = per_worker // _IDS_GROUP
    vregs = dim // _LANES
    npass = seq // _PASS
    chunks = [(p * _PASS, _PASS) for p in range(npass)]

    mesh = plsc.VectorSubcoreMesh(core_axis_name="c", subcore_axis_name="s")

    @functools.partial(
        pl.kernel,
        out_type=jax.ShapeDtypeStruct((batch, dim), jnp.float32),
        mesh=mesh,
        scratch_types=[
            pltpu.VMEM((2, _IDS_GROUP * seq), jnp.int32),
            pltpu.VMEM((_DEPTH, _PASS, dim), jnp.float32),
            pltpu.VMEM((per_worker, dim), jnp.float32),
            pltpu.SemaphoreType.DMA,
            pltpu.SemaphoreType.DMA,
        ],
        compiler_params=pltpu.CompilerParams(use_tc_tiling_on_sc=False),
    )
    def body(ids_hbm, table_hbm, out_hbm, ids_v, rows_v, out_v, sem_g, sem_i):
        wid = lax.axis_index("s") * _NUM_CORES + lax.axis_index("c")
        base = wid * per_worker

        def ids_copy(g, par):
            grp = base + g * _IDS_GROUP
            return pltpu.make_async_copy(
                ids_hbm.at[pl.ds(grp * seq, _IDS_GROUP * seq)],
                ids_v.at[par],
                sem_i,
            )

        def issue_gathers(s, ipar, rpar):
            return [
                pltpu.async_copy(
                    table_hbm.at[ids_v.at[ipar, pl.ds(s * seq + off, n)]],
                    rows_v.at[rpar],
                    sem_g,
                    add=True,
                )
                for off, n in chunks
            ]

        def reduce_store(s, rpar, row):
            zero = jnp.zeros((_LANES,), jnp.float32)

            def red_body(t, a):
                t8 = t * 8
                for dt in range(8):
                    vs = []
                    for k in range(vregs):
                        sl = pl.ds(k * _LANES, _LANES)
                        vs.append(rows_v[rpar, t8 + dt, sl])
                        rows_v[rpar, t8 + dt, sl] = zero
                    a = tuple(a[k] + vs[k] for k in range(vregs))
                return a

            acc = lax.fori_loop(0, _PASS // 8, red_body, (zero,) * vregs)

            scale = jnp.float32(1.0 / seq)
            for k in range(vregs):
                out_v[row, pl.ds(k * _LANES, _LANES)] = acc[k] * scale

        zero16 = jnp.zeros((_LANES,), jnp.float32)

        def zero_body(t, _):
            for d in range(_DEPTH):
                for k in range(vregs):
                    rows_v[d, t, pl.ds(k * _LANES, _LANES)] = zero16
            return 0

        lax.fori_loop(0, _PASS, zero_body, 0)
        ids_copy(0, 0).start()

        def two_groups(h, _):
            for par in (0, 1):
                g = 2 * h + par
                ids_copy(g, par).wait()
                @pl.when(g + 1 < groups)
                def _():
                    ids_copy(g + 1, 1 - par).start()

                ahead = _DEPTH - 1
                cps = {
                    s: issue_gathers(s, par, s % _DEPTH)
                    for s in range(min(ahead, _IDS_GROUP))
                }
                for s in range(_IDS_GROUP):
                    if s + ahead < _IDS_GROUP:
                        cps[s + ahead] = issue_gathers(
                            s + ahead, par, (s + ahead) % _DEPTH
                        )
                    for cp in cps.pop(s):
                        cp.wait()
                    reduce_store(s, s % _DEPTH, g * _IDS_GROUP + s)
            return 0

        lax.fori_loop(0, groups // 2, two_groups, 0)
        pltpu.sync_copy(out_v, out_hbm.at[pl.ds(base, per_worker)])

    return body(ids_flat, table)


def _classifier_body(x_ref, w_ref, b_ref, o_ref):
    o_ref[...] = (
        lax.dot_general(
            x_ref[...],
            w_ref[...],
            (((1,), (1,)), ((), ())),
            preferred_element_type=jnp.float32,
        )
        + b_ref[...]
    )


def kernel(input_ids, emb_table, W, b):
    batch, seq = input_ids.shape
    _, dim = emb_table.shape
    ncls = W.shape[0]

    ids_flat = input_ids.reshape(-1).astype(jnp.int32)
    x = _emb_mean(ids_flat, emb_table, batch=batch, seq=seq, dim=dim)
    logits = pl.pallas_call(
        _classifier_body,
        out_shape=jax.ShapeDtypeStruct((batch, ncls), jnp.float32),
    )(x, W, b.reshape(1, ncls))
    return (logits, x)
